# Initial kernel scaffold; baseline (speedup 1.0000x reference)
import jax
import jax.numpy as jnp
from jax.experimental import pallas as pl


def kernel(x, adj, gc1_weight, gc1_bias, gc2_weight, gc2_bias, fc3_weight, fc3_bias, encoder_type):
    raise NotImplementedError("write your pallas kernel here")


# speedup vs baseline: 1.1501x; 1.1501x over previous
import functools

import jax
import jax.numpy as jnp
from jax.experimental import pallas as pl
from jax.experimental.pallas import tpu as pltpu


def _k1_body(x_ref, w_ref, gate_ref, o_ref):
    g = gate_ref[0, 0]
    o_ref[...] = jnp.dot(x_ref[...], w_ref[...],
                         preferred_element_type=jnp.float32) * g


def _k2_body(adj_ref, s1_ref, b1_ref, w2_ref, o_ref):
    acc = jnp.dot(adj_ref[...], s1_ref[...],
                  preferred_element_type=jnp.float32)
    h = jnp.maximum(acc + b1_ref[...], 0.0)
    o_ref[...] = jnp.dot(h, w2_ref[...], preferred_element_type=jnp.float32)


def _k3_body(adj_ref, s2_ref, b2_ref, w3t_ref, b3_ref, logsm_ref, out_ref):
    x2 = jnp.dot(adj_ref[...], s2_ref[...],
                 preferred_element_type=jnp.float32) + b2_ref[...]
    m = jnp.max(x2, axis=1, keepdims=True)
    lse = m + jnp.log(jnp.sum(jnp.exp(x2 - m), axis=1, keepdims=True))
    logsm_ref[...] = x2 - lse
    r = jnp.maximum(x2, 0.0)
    out_ref[...] = jnp.dot(r, w3t_ref[...],
                           preferred_element_type=jnp.float32) + b3_ref[...]


@functools.partial(jax.jit, static_argnames=())
def _run(x, adj, w1, b1, w2, b2, w3, b3, encoder_type):
    N, F = x.shape
    H = w1.shape[1]
    C = w2.shape[1]
    O = w3.shape[0]

    gate = jnp.asarray(jnp.equal(encoder_type, 0), x.dtype).reshape(1, 1)
    b1r = b1.reshape(1, H)
    b2r = b2.reshape(1, C)
    b3r = b3.reshape(1, O)
    w3t = w3.T

    BR = 512
    nblk = N // BR

    s1 = pl.pallas_call(
        _k1_body,
        grid=(nblk,),
        in_specs=[
            pl.BlockSpec((BR, F), lambda i: (i, 0)),
            pl.BlockSpec((F, H), lambda i: (0, 0)),
            pl.BlockSpec(memory_space=pltpu.SMEM),
        ],
        out_specs=pl.BlockSpec((BR, H), lambda i: (i, 0)),
        out_shape=jax.ShapeDtypeStruct((N, H), jnp.float32),
    )(x, w1, gate)

    s2 = pl.pallas_call(
        _k2_body,
        grid=(nblk,),
        in_specs=[
            pl.BlockSpec((BR, N), lambda i: (i, 0)),
            pl.BlockSpec((N, H), lambda i: (0, 0)),
            pl.BlockSpec((1, H), lambda i: (0, 0)),
            pl.BlockSpec((H, C), lambda i: (0, 0)),
        ],
        out_specs=pl.BlockSpec((BR, C), lambda i: (i, 0)),
        out_shape=jax.ShapeDtypeStruct((N, C), jnp.float32),
    )(adj, s1, b1r, w2)

    logsm, out = pl.pallas_call(
        _k3_body,
        grid=(nblk,),
        in_specs=[
            pl.BlockSpec((BR, N), lambda i: (i, 0)),
            pl.BlockSpec((N, C), lambda i: (0, 0)),
            pl.BlockSpec((1, C), lambda i: (0, 0)),
            pl.BlockSpec((C, O), lambda i: (0, 0)),
            pl.BlockSpec((1, O), lambda i: (0, 0)),
        ],
        out_specs=[
            pl.BlockSpec((BR, C), lambda i: (i, 0)),
            pl.BlockSpec((BR, O), lambda i: (i, 0)),
        ],
        out_shape=[
            jax.ShapeDtypeStruct((N, C), jnp.float32),
            jax.ShapeDtypeStruct((N, O), jnp.float32),
        ],
    )(adj, s2, b2r, w3t, b3r)

    return logsm, out


def kernel(x, adj, gc1_weight, gc1_bias, gc2_weight, gc2_bias,
           fc3_weight, fc3_bias, encoder_type):
    return _run(x, adj, gc1_weight, gc1_bias, gc2_weight, gc2_bias,
                fc3_weight, fc3_bias, encoder_type)
